# baseline (speedup 1.0000x reference)
import functools

import jax
import jax.numpy as jnp
from jax import lax
from jax.experimental import pallas as pl
from jax.experimental.pallas import tpu as pltpu
from jax.experimental.pallas import tpu_sc as plsc

N = 10000
E = 320000
IN_NF = 128
IN_EF = 16
OUT_NF = 128

NUM_CORES = 2
NUM_TILES = 16
NUM_WORKERS = NUM_CORES * NUM_TILES
CHUNK = 128
TOTAL_ROWS = E // CHUNK
BASE_ROWS = TOTAL_ROWS // NUM_WORKERS
LEFTOVER = TOTAL_ROWS - BASE_ROWS * NUM_WORKERS
NP = 10112
ROWS_PER_TILE = NP // NUM_TILES
RE_CHUNK = CHUNK * 16


def _lane_perm(v, idx):
    dn = lax.GatherDimensionNumbers(offset_dims=(), collapsed_slice_dims=(0,),
                                    start_index_map=(0,))
    return lax.gather(v, idx[:, None], dn, slice_sizes=(1,),
                      mode=lax.GatherScatterMode.PROMISE_IN_BOUNDS)


def _edge_sc_kernel(ts_h, td_h, rea_h, reb_h, ei_h, zz_h, w0_h, b0_h,
                    sd_h, ss_h,
                    isf, idf, isx2, idx2, ga2, gb2, rba2, rbb2, pd, ps,
                    w0s, b0s, sdacc, ssacc,
                    sga0, sga1, sgb0, sgb1, sre0, sre1):
    f32 = jnp.float32
    cid = lax.axis_index("c")
    sid = lax.axis_index("s")
    wid = sid * NUM_CORES + cid
    row0 = sid * ROWS_PER_TILE
    crow0 = wid * BASE_ROWS

    pltpu.sync_copy(zz_h.at[pl.ds(row0, ROWS_PER_TILE)],
                    sdacc.at[pl.ds(row0, ROWS_PER_TILE)])
    pltpu.sync_copy(zz_h.at[pl.ds(row0, ROWS_PER_TILE)],
                    ssacc.at[pl.ds(row0, ROWS_PER_TILE)])
    pltpu.sync_copy(w0_h, w0s)
    pltpu.sync_copy(b0_h, b0s)
    pltpu.sync_copy(ei_h.at[0, pl.ds(crow0 * CHUNK, BASE_ROWS * CHUNK)],
                    isf.at[pl.ds(0, BASE_ROWS * CHUNK)])
    pltpu.sync_copy(ei_h.at[1, pl.ds(crow0 * CHUNK, BASE_ROWS * CHUNK)],
                    idf.at[pl.ds(0, BASE_ROWS * CHUNK)])

    @pl.when(wid < LEFTOVER)
    def _():
        eoff = (TOTAL_ROWS - LEFTOVER + wid) * CHUNK
        pltpu.sync_copy(ei_h.at[0, pl.ds(eoff, CHUNK)],
                        isf.at[pl.ds(BASE_ROWS * CHUNK, CHUNK)])
        pltpu.sync_copy(ei_h.at[1, pl.ds(eoff, CHUNK)],
                        idf.at[pl.ds(BASE_ROWS * CHUNK, CHUNK)])

    plsc.subcore_barrier()

    w0r = w0s[...]
    b0r = b0s[...]
    lane = lax.broadcasted_iota(jnp.int32, (16,), 0)
    one = jnp.full((16,), 1.0, f32)
    zero = jnp.full((16,), 0.0, f32)
    cntv = jnp.where(lane == 0, one, zero)
    px1 = jnp.bitwise_xor(lane, 1)
    px2 = jnp.bitwise_xor(lane, 2)
    px4 = jnp.bitwise_xor(lane, 4)
    px8 = jnp.bitwise_xor(lane, 8)
    sems = ((sga0, sgb0, sre0), (sga1, sgb1, sre1))
    bufs = ((ga2.at[0], gb2.at[0], rba2.at[0], rbb2.at[0], isx2.at[0], idx2.at[0]),
            (ga2.at[1], gb2.at[1], rba2.at[1], rbb2.at[1], isx2.at[1], idx2.at[1]))

    def _stage_idx(c, b):
        _, _, _, _, is_b, id_b = bufs[b]
        coff = pl.multiple_of(c * CHUNK, CHUNK)
        for k in range(CHUNK // 16):
            is_b[0, pl.ds(16 * k, 16)] = isf[pl.ds(coff + 16 * k, 16)]
            id_b[0, pl.ds(16 * k, 16)] = idf[pl.ds(coff + 16 * k, 16)]

    def _descs(c, b):
        ga_b, gb_b, rba_b, rbb_b, is_b, id_b = bufs[b]
        sga, sgb, sre = sems[b]
        roff = pl.multiple_of((crow0 + c) * RE_CHUNK, RE_CHUNK)
        return (pltpu.make_async_copy(ts_h.at[is_b.at[0]], ga_b, sga),
                pltpu.make_async_copy(td_h.at[id_b.at[0]], gb_b, sgb),
                pltpu.make_async_copy(rea_h.at[pl.ds(roff, RE_CHUNK)], rba_b, sre),
                pltpu.make_async_copy(reb_h.at[pl.ds(roff, RE_CHUNK)], rbb_b, sre))

    def _fire(c, b):
        _stage_idx(c, b)
        for d in _descs(c, b):
            d.start()

    def _wait(c, b):
        for d in _descs(c, b):
            d.wait()

    def _compute(b):
        ga_b, gb_b, rba_b, rbb_b, _, _ = bufs[b]

        def pair_body(t, tc):
            base = pl.multiple_of(t * 128, 128)
            for half in range(2):
                rb_b = rba_b if half == 0 else rbb_b
                for eo in range(4):
                    e = t * 8 + half * 4 + eo
                    a0 = ga_b[e, pl.ds(0, 16)]
                    a1 = ga_b[e, pl.ds(16, 16)]
                    c0 = gb_b[e, pl.ds(0, 16)]
                    c1 = gb_b[e, pl.ds(16, 16)]
                    r0 = rb_b[pl.ds(base + eo * 32, 16)]
                    r1 = rb_b[pl.ds(base + eo * 32 + 16, 16)]
                    h1 = a0 + c0 + r0
                    g1 = jnp.where(h1 > 0, h1, 0.2 * h1)
                    h2 = a1 + c1 + r1
                    g2 = jnp.where(h2 > 0, h2, 0.2 * h2)
                    sv = g2 * w0r
                    sv = sv + _lane_perm(sv, px1)
                    sv = sv + _lane_perm(sv, px2)
                    sv = sv + _lane_perm(sv, px4)
                    sv = sv + _lane_perm(sv, px8)
                    kv = 1.0 / (1.0 + jnp.exp(-(sv + b0r)))
                    u = kv * g2
                    tail = jnp.where(lane == 0, kv,
                                     jnp.where(lane == 1, one, zero))
                    pd[e, pl.ds(0, 16)] = g1
                    ps[e, pl.ds(0, 16)] = u
                    ps[e, pl.ds(16, 16)] = tail
            return tc

        lax.fori_loop(0, CHUNK // 8, pair_body, 0)

    def _scatter(b):
        _, _, _, _, is_b, id_b = bufs[b]
        pltpu.sync_copy(pd, sdacc.at[id_b.at[0]], add=True)
        pltpu.sync_copy(ps, ssacc.at[is_b.at[0]], add=True)

    def _cnt_init(r, carry):
        pd[r, pl.ds(16, 16)] = cntv
        return carry

    lax.fori_loop(0, CHUNK, _cnt_init, 0)

    _fire(0, 0)

    def body(i, carry):
        c0 = 2 * i
        c1 = c0 + 1
        _fire(c1, 1)
        _wait(c0, 0)
        _compute(0)
        _scatter(0)

        @pl.when(i < BASE_ROWS // 2 - 1)
        def _():
            _fire(c0 + 2, 0)

        _wait(c1, 1)
        _compute(1)
        _scatter(1)
        return carry

    lax.fori_loop(0, BASE_ROWS // 2, body, 0)

    @pl.when(wid < LEFTOVER)
    def _():
        _stage_idx(BASE_ROWS, 0)
        erow = TOTAL_ROWS - LEFTOVER + wid
        ga_b, gb_b, rba_b, rbb_b, is_b, id_b = bufs[0]
        sga, sgb, sre = sems[0]
        roff = erow * RE_CHUNK
        ds = (pltpu.make_async_copy(ts_h.at[is_b.at[0]], ga_b, sga),
              pltpu.make_async_copy(td_h.at[id_b.at[0]], gb_b, sgb),
              pltpu.make_async_copy(rea_h.at[pl.ds(roff, RE_CHUNK)], rba_b, sre),
              pltpu.make_async_copy(reb_h.at[pl.ds(roff, RE_CHUNK)], rbb_b, sre))
        for d in ds:
            d.start()
        for d in ds:
            d.wait()
        _compute(0)
        _scatter(0)

    plsc.subcore_barrier()
    pltpu.sync_copy(sdacc.at[pl.ds(row0, ROWS_PER_TILE)],
                    sd_h.at[cid, pl.ds(row0, ROWS_PER_TILE)])
    pltpu.sync_copy(ssacc.at[pl.ds(row0, ROWS_PER_TILE)],
                    ss_h.at[cid, pl.ds(row0, ROWS_PER_TILE)])


def _tables_body(nf_ref, ws_ref, wd_ref, ts_ref, td_ref):
    x = nf_ref[...]
    ts_ref[...] = jnp.dot(x, ws_ref[...], preferred_element_type=jnp.float32)
    td_ref[...] = jnp.dot(x, wd_ref[...], preferred_element_type=jnp.float32)


def _re_body(nef8_ref, w8a_ref, w8b_ref, b8_ref, rea_ref, reb_ref):
    x = nef8_ref[...]
    rea_ref[...] = (jnp.dot(x, w8a_ref[...],
                            preferred_element_type=jnp.float32) + b8_ref[...])
    reb_ref[...] = (jnp.dot(x, w8b_ref[...],
                            preferred_element_type=jnp.float32) + b8_ref[...])


def _fin_body(sd_ref, ss_ref, w2o_ref, b2o_ref, w2f_ref, b2f_ref,
              w1r_ref, b1r_ref, w2r_ref, b2r_ref, out_ref):
    f32 = jnp.float32
    sd = sd_ref[0] + sd_ref[1]
    ss = ss_ref[0] + ss_ref[1]
    s1 = sd[:, 0:16]
    cntd = sd[:, 16:17]
    new_nf = jnp.dot(s1, w2o_ref[...], preferred_element_type=f32) + cntd * b2o_ref[...]
    s2 = ss[:, 0:16]
    ks = ss[:, 16:17]
    cnts = ss[:, 17:18]
    nfo12 = jnp.dot(s2, w2f_ref[...], preferred_element_type=f32) + ks * b2f_ref[...]
    nfo2 = nfo12[:, 8:16] / jnp.maximum(cnts, 1.0)
    hin = jnp.concatenate([new_nf, nfo12[:, 0:8], nfo2], axis=1)
    h = jnp.dot(hin, w1r_ref[...], preferred_element_type=f32) + b1r_ref[...]
    h = jnp.where(h > 0, h, 0.2 * h)
    red = jnp.dot(h, w2r_ref[...], preferred_element_type=f32) + b2r_ref[...]
    out_ref[...] = jnp.where(cnts > 0, red, new_nf)


def kernel(nf, edge_index, nef,
           W1_o2i, b1_o2i, W2_o2i, b2_o2i,
           W1_i2o, b1_i2o, W2_i2o, b2_i2o,
           W1_red, b1_red, W2_red, b2_red):
    f32 = jnp.float32
    i32 = jnp.int32

    ws = jnp.concatenate([W1_o2i[:IN_NF], W1_i2o[IN_NF:2 * IN_NF]], axis=1)
    wd = jnp.concatenate([W1_o2i[IN_NF:2 * IN_NF], W1_i2o[:IN_NF]], axis=1)
    wre = jnp.concatenate([W1_o2i[2 * IN_NF:], W1_i2o[2 * IN_NF:]], axis=1)
    bre = jnp.concatenate([b1_o2i, b1_i2o]).reshape(1, 32)
    wk = jnp.kron(jnp.eye(4, dtype=f32), wre)
    zpad = jnp.zeros((64, 128), f32)
    w8a = jnp.concatenate([wk, zpad], axis=0)
    w8b = jnp.concatenate([zpad, wk], axis=0)
    b8 = jnp.tile(bre, (1, 4))
    nef8 = nef.reshape(E // 8, 8 * IN_EF)
    w0v = W2_i2o[:, 0]
    b0v = jnp.full((16,), 1.0, f32) * b2_i2o[0]
    zeros_acc = jnp.zeros((NP, 32), f32)

    ts, td = pl.pallas_call(
        _tables_body,
        out_shape=(jax.ShapeDtypeStruct((N, 32), f32),
                   jax.ShapeDtypeStruct((N, 32), f32)),
    )(nf, ws, wd)

    BLK8 = 4000
    rea, reb = pl.pallas_call(
        _re_body,
        grid=(E // 8 // BLK8,),
        in_specs=[pl.BlockSpec((BLK8, 128), lambda i: (i, 0)),
                  pl.BlockSpec((128, 128), lambda i: (0, 0)),
                  pl.BlockSpec((128, 128), lambda i: (0, 0)),
                  pl.BlockSpec((1, 128), lambda i: (0, 0))],
        out_specs=(pl.BlockSpec((BLK8, 128), lambda i: (i, 0)),
                   pl.BlockSpec((BLK8, 128), lambda i: (i, 0))),
        out_shape=(jax.ShapeDtypeStruct((E // 8, 128), f32),
                   jax.ShapeDtypeStruct((E // 8, 128), f32)),
    )(nef8, w8a, w8b, b8)
    rea1 = rea.reshape(E * 16)
    reb1 = reb.reshape(E * 16)

    mesh = plsc.VectorSubcoreMesh(core_axis_name="c", subcore_axis_name="s")
    edge_fn = functools.partial(
        pl.kernel,
        out_type=(jax.ShapeDtypeStruct((NUM_CORES, NP, 32), f32),
                  jax.ShapeDtypeStruct((NUM_CORES, NP, 32), f32)),
        mesh=mesh,
        scratch_types=[
            pltpu.VMEM(((BASE_ROWS + 1) * CHUNK,), i32),
            pltpu.VMEM(((BASE_ROWS + 1) * CHUNK,), i32),
            pltpu.VMEM((2, 1, CHUNK), i32),
            pltpu.VMEM((2, 1, CHUNK), i32),
            pltpu.VMEM((2, CHUNK, 32), f32),
            pltpu.VMEM((2, CHUNK, 32), f32),
            pltpu.VMEM((2, RE_CHUNK), f32),
            pltpu.VMEM((2, RE_CHUNK), f32),
            pltpu.VMEM((CHUNK, 32), f32),
            pltpu.VMEM((CHUNK, 32), f32),
            pltpu.VMEM((16,), f32),
            pltpu.VMEM((16,), f32),
            pltpu.VMEM_SHARED((NP, 32), f32),
            pltpu.VMEM_SHARED((NP, 32), f32),
            pltpu.SemaphoreType.DMA,
            pltpu.SemaphoreType.DMA,
            pltpu.SemaphoreType.DMA,
            pltpu.SemaphoreType.DMA,
            pltpu.SemaphoreType.DMA,
            pltpu.SemaphoreType.DMA,
        ],
        compiler_params=pltpu.CompilerParams(use_tc_tiling_on_sc=False),
    )(_edge_sc_kernel)
    sd_part, ss_part = edge_fn(ts, td, rea1, reb1, edge_index, zeros_acc,
                               w0v, b0v)

    b2o = b2_o2i.reshape(1, OUT_NF)
    w2f = W2_i2o[:, 1:17]
    b2f = b2_i2o[1:17].reshape(1, 16)
    b1r = b1_red.reshape(1, 16)
    b2r = b2_red.reshape(1, OUT_NF)
    RBLK = 2000
    out = pl.pallas_call(
        _fin_body,
        grid=(N // RBLK,),
        in_specs=[pl.BlockSpec((NUM_CORES, RBLK, 32), lambda i: (0, i, 0)),
                  pl.BlockSpec((NUM_CORES, RBLK, 32), lambda i: (0, i, 0)),
                  pl.BlockSpec((16, OUT_NF), lambda i: (0, 0)),
                  pl.BlockSpec((1, OUT_NF), lambda i: (0, 0)),
                  pl.BlockSpec((16, 16), lambda i: (0, 0)),
                  pl.BlockSpec((1, 16), lambda i: (0, 0)),
                  pl.BlockSpec((144, 16), lambda i: (0, 0)),
                  pl.BlockSpec((1, 16), lambda i: (0, 0)),
                  pl.BlockSpec((16, OUT_NF), lambda i: (0, 0)),
                  pl.BlockSpec((1, OUT_NF), lambda i: (0, 0))],
        out_specs=pl.BlockSpec((RBLK, OUT_NF), lambda i: (i, 0)),
        out_shape=jax.ShapeDtypeStruct((N, OUT_NF), f32),
    )(sd_part, ss_part, W2_o2i, b2o, w2f, b2f, W1_red, b1r, W2_red, b2r)
    return out


# speedup vs baseline: 1.0079x; 1.0079x over previous
import functools

import jax
import jax.numpy as jnp
from jax import lax
from jax.experimental import pallas as pl
from jax.experimental.pallas import tpu as pltpu
from jax.experimental.pallas import tpu_sc as plsc

N = 10000
E = 320000
IN_NF = 128
IN_EF = 16
OUT_NF = 128

NUM_CORES = 2
NUM_TILES = 16
NUM_WORKERS = NUM_CORES * NUM_TILES
CHUNK = 128
TOTAL_ROWS = E // CHUNK
BASE_ROWS = TOTAL_ROWS // NUM_WORKERS
LEFTOVER = TOTAL_ROWS - BASE_ROWS * NUM_WORKERS
NP = 10112
ROWS_PER_TILE = NP // NUM_TILES
RE_CHUNK = CHUNK * 16


def _lane_perm(v, idx):
    dn = lax.GatherDimensionNumbers(offset_dims=(), collapsed_slice_dims=(0,),
                                    start_index_map=(0,))
    return lax.gather(v, idx[:, None], dn, slice_sizes=(1,),
                      mode=lax.GatherScatterMode.PROMISE_IN_BOUNDS)


def _edge_sc_kernel(ts_h, td_h, rea_h, reb_h, ei_h, zz_h, w0_h, b0_h,
                    sd_h, ss_h,
                    isv2, idv2, ga2, gb2, rba0, rba1, rbb0, rbb1, pd, ps,
                    w0s, b0s, sdacc, ssacc,
                    sga0, sga1, sgb0, sgb1, sre0, sre1, sidx):
    f32 = jnp.float32
    cid = lax.axis_index("c")
    sid = lax.axis_index("s")
    wid = sid * NUM_CORES + cid
    row0 = sid * ROWS_PER_TILE
    crow0 = wid * BASE_ROWS

    pltpu.sync_copy(zz_h.at[pl.ds(row0, ROWS_PER_TILE)],
                    sdacc.at[pl.ds(row0, ROWS_PER_TILE)])
    pltpu.sync_copy(zz_h.at[pl.ds(row0, ROWS_PER_TILE)],
                    ssacc.at[pl.ds(row0, ROWS_PER_TILE)])
    pltpu.sync_copy(w0_h, w0s)
    pltpu.sync_copy(b0_h, b0s)

    def _idx_descs(c, carry):
        eb = pl.multiple_of((crow0 + c) * CHUNK, CHUNK)
        pltpu.make_async_copy(ei_h.at[0, pl.ds(eb, CHUNK)],
                              isv2.at[c], sidx).start()
        pltpu.make_async_copy(ei_h.at[1, pl.ds(eb, CHUNK)],
                              idv2.at[c], sidx).start()
        return carry

    def _idx_waits(c, carry):
        eb = pl.multiple_of((crow0 + c) * CHUNK, CHUNK)
        pltpu.make_async_copy(ei_h.at[0, pl.ds(eb, CHUNK)],
                              isv2.at[c], sidx).wait()
        pltpu.make_async_copy(ei_h.at[1, pl.ds(eb, CHUNK)],
                              idv2.at[c], sidx).wait()
        return carry

    lax.fori_loop(0, BASE_ROWS, _idx_descs, 0)

    @pl.when(wid < LEFTOVER)
    def _():
        erow = TOTAL_ROWS - LEFTOVER + wid
        eb = erow * CHUNK
        pltpu.make_async_copy(ei_h.at[0, pl.ds(eb, CHUNK)],
                              isv2.at[BASE_ROWS], sidx).start()
        pltpu.make_async_copy(ei_h.at[1, pl.ds(eb, CHUNK)],
                              idv2.at[BASE_ROWS], sidx).start()
        pltpu.make_async_copy(ei_h.at[0, pl.ds(eb, CHUNK)],
                              isv2.at[BASE_ROWS], sidx).wait()
        pltpu.make_async_copy(ei_h.at[1, pl.ds(eb, CHUNK)],
                              idv2.at[BASE_ROWS], sidx).wait()

    lax.fori_loop(0, BASE_ROWS, _idx_waits, 0)
    plsc.subcore_barrier()

    w0r = w0s[...]
    b0r = b0s[...]
    lane = lax.broadcasted_iota(jnp.int32, (16,), 0)
    one = jnp.full((16,), 1.0, f32)
    zero = jnp.full((16,), 0.0, f32)
    cntv = jnp.where(lane == 0, one, zero)
    px1 = jnp.bitwise_xor(lane, 1)
    px2 = jnp.bitwise_xor(lane, 2)
    px4 = jnp.bitwise_xor(lane, 4)
    px8 = jnp.bitwise_xor(lane, 8)
    sems = ((sga0, sgb0, sre0), (sga1, sgb1, sre1))
    bufs = ((ga2.at[0], gb2.at[0], rba0, rbb0), (ga2.at[1], gb2.at[1], rba1, rbb1))

    def _descs(c, b):
        ga_b, gb_b, rba_b, rbb_b = bufs[b]
        sga, sgb, sre = sems[b]
        rrow = pl.multiple_of((crow0 + c) * 16, 16)
        return (pltpu.make_async_copy(ts_h.at[isv2.at[c]], ga_b, sga),
                pltpu.make_async_copy(td_h.at[idv2.at[c]], gb_b, sgb),
                pltpu.make_async_copy(rea_h.at[pl.ds(rrow, 16)], rba_b, sre),
                pltpu.make_async_copy(reb_h.at[pl.ds(rrow, 16)], rbb_b, sre))

    def _fire(c, b):
        for d in _descs(c, b):
            d.start()

    def _wait(c, b):
        for d in _descs(c, b):
            d.wait()

    def _compute(b):
        ga_b, gb_b, rba_b, rbb_b = bufs[b]

        def pair_body(t, tc):
            for half in range(2):
                rb_b = rba_b if half == 0 else rbb_b
                for eo in range(4):
                    e = t * 8 + half * 4 + eo
                    a0 = ga_b[e, pl.ds(0, 16)]
                    a1 = ga_b[e, pl.ds(16, 16)]
                    c0 = gb_b[e, pl.ds(0, 16)]
                    c1 = gb_b[e, pl.ds(16, 16)]
                    r0 = rb_b[t, pl.ds(eo * 32, 16)]
                    r1 = rb_b[t, pl.ds(eo * 32 + 16, 16)]
                    h1 = a0 + c0 + r0
                    g1 = jnp.where(h1 > 0, h1, 0.2 * h1)
                    h2 = a1 + c1 + r1
                    g2 = jnp.where(h2 > 0, h2, 0.2 * h2)
                    sv = g2 * w0r
                    sv = sv + _lane_perm(sv, px1)
                    sv = sv + _lane_perm(sv, px2)
                    sv = sv + _lane_perm(sv, px4)
                    sv = sv + _lane_perm(sv, px8)
                    kv = 1.0 / (1.0 + jnp.exp(-(sv + b0r)))
                    u = kv * g2
                    tail = jnp.where(lane == 0, kv,
                                     jnp.where(lane == 1, one, zero))
                    pd[e, pl.ds(0, 16)] = g1
                    ps[e, pl.ds(0, 16)] = u
                    ps[e, pl.ds(16, 16)] = tail
            return tc

        lax.fori_loop(0, CHUNK // 8, pair_body, 0)

    def _scatter(c):
        pltpu.sync_copy(pd, sdacc.at[idv2.at[c]], add=True)
        pltpu.sync_copy(ps, ssacc.at[isv2.at[c]], add=True)

    def _cnt_init(r, carry):
        pd[r, pl.ds(16, 16)] = cntv
        return carry

    lax.fori_loop(0, CHUNK, _cnt_init, 0)

    _fire(0, 0)

    def body(i, carry):
        c0 = 2 * i
        c1 = c0 + 1
        _fire(c1, 1)
        _wait(c0, 0)
        _compute(0)
        _scatter(c0)

        @pl.when(i < BASE_ROWS // 2 - 1)
        def _():
            _fire(c0 + 2, 0)

        _wait(c1, 1)
        _compute(1)
        _scatter(c1)
        return carry

    lax.fori_loop(0, BASE_ROWS // 2, body, 0)

    @pl.when(wid < LEFTOVER)
    def _():
        erow = TOTAL_ROWS - LEFTOVER + wid
        ga_b, gb_b, rba_b, rbb_b = bufs[0]
        sga, sgb, sre = sems[0]
        rrow = erow * 16
        ds = (pltpu.make_async_copy(ts_h.at[isv2.at[BASE_ROWS]], ga_b, sga),
              pltpu.make_async_copy(td_h.at[idv2.at[BASE_ROWS]], gb_b, sgb),
              pltpu.make_async_copy(rea_h.at[pl.ds(rrow, 16)], rba_b, sre),
              pltpu.make_async_copy(reb_h.at[pl.ds(rrow, 16)], rbb_b, sre))
        for d in ds:
            d.start()
        for d in ds:
            d.wait()
        _compute(0)
        _scatter(BASE_ROWS)

    plsc.subcore_barrier()
    pltpu.sync_copy(sdacc.at[pl.ds(row0, ROWS_PER_TILE)],
                    sd_h.at[cid, pl.ds(row0, ROWS_PER_TILE)])
    pltpu.sync_copy(ssacc.at[pl.ds(row0, ROWS_PER_TILE)],
                    ss_h.at[cid, pl.ds(row0, ROWS_PER_TILE)])


def _tables_body(nf_ref, ws_ref, wd_ref, ts_ref, td_ref):
    x = nf_ref[...]
    ts_ref[...] = jnp.dot(x, ws_ref[...], preferred_element_type=jnp.float32)
    td_ref[...] = jnp.dot(x, wd_ref[...], preferred_element_type=jnp.float32)


def _re_body(nef8_ref, w8a_ref, w8b_ref, b8_ref, rea_ref, reb_ref):
    x = nef8_ref[...]
    rea_ref[...] = (jnp.dot(x, w8a_ref[...],
                            preferred_element_type=jnp.float32) + b8_ref[...])
    reb_ref[...] = (jnp.dot(x, w8b_ref[...],
                            preferred_element_type=jnp.float32) + b8_ref[...])


def _fin_body(sd_ref, ss_ref, w2o_ref, b2o_ref, w2f_ref, b2f_ref,
              w1r_ref, b1r_ref, w2r_ref, b2r_ref, out_ref):
    f32 = jnp.float32
    sd = sd_ref[0] + sd_ref[1]
    ss = ss_ref[0] + ss_ref[1]
    s1 = sd[:, 0:16]
    cntd = sd[:, 16:17]
    new_nf = jnp.dot(s1, w2o_ref[...], preferred_element_type=f32) + cntd * b2o_ref[...]
    s2 = ss[:, 0:16]
    ks = ss[:, 16:17]
    cnts = ss[:, 17:18]
    nfo12 = jnp.dot(s2, w2f_ref[...], preferred_element_type=f32) + ks * b2f_ref[...]
    nfo2 = nfo12[:, 8:16] / jnp.maximum(cnts, 1.0)
    hin = jnp.concatenate([new_nf, nfo12[:, 0:8], nfo2], axis=1)
    h = jnp.dot(hin, w1r_ref[...], preferred_element_type=f32) + b1r_ref[...]
    h = jnp.where(h > 0, h, 0.2 * h)
    red = jnp.dot(h, w2r_ref[...], preferred_element_type=f32) + b2r_ref[...]
    out_ref[...] = jnp.where(cnts > 0, red, new_nf)


def kernel(nf, edge_index, nef,
           W1_o2i, b1_o2i, W2_o2i, b2_o2i,
           W1_i2o, b1_i2o, W2_i2o, b2_i2o,
           W1_red, b1_red, W2_red, b2_red):
    f32 = jnp.float32
    i32 = jnp.int32

    ws = jnp.concatenate([W1_o2i[:IN_NF], W1_i2o[IN_NF:2 * IN_NF]], axis=1)
    wd = jnp.concatenate([W1_o2i[IN_NF:2 * IN_NF], W1_i2o[:IN_NF]], axis=1)
    wre = jnp.concatenate([W1_o2i[2 * IN_NF:], W1_i2o[2 * IN_NF:]], axis=1)
    bre = jnp.concatenate([b1_o2i, b1_i2o]).reshape(1, 32)
    wk = jnp.kron(jnp.eye(4, dtype=f32), wre)
    zpad = jnp.zeros((64, 128), f32)
    w8a = jnp.concatenate([wk, zpad], axis=0)
    w8b = jnp.concatenate([zpad, wk], axis=0)
    b8 = jnp.tile(bre, (1, 4))
    nef8 = nef.reshape(E // 8, 8 * IN_EF)
    w0v = W2_i2o[:, 0]
    b0v = jnp.full((16,), 1.0, f32) * b2_i2o[0]
    zeros_acc = jnp.zeros((NP, 32), f32)

    ts, td = pl.pallas_call(
        _tables_body,
        out_shape=(jax.ShapeDtypeStruct((N, 32), f32),
                   jax.ShapeDtypeStruct((N, 32), f32)),
    )(nf, ws, wd)

    BLK8 = 4000
    rea, reb = pl.pallas_call(
        _re_body,
        grid=(E // 8 // BLK8,),
        in_specs=[pl.BlockSpec((BLK8, 128), lambda i: (i, 0)),
                  pl.BlockSpec((128, 128), lambda i: (0, 0)),
                  pl.BlockSpec((128, 128), lambda i: (0, 0)),
                  pl.BlockSpec((1, 128), lambda i: (0, 0))],
        out_specs=(pl.BlockSpec((BLK8, 128), lambda i: (i, 0)),
                   pl.BlockSpec((BLK8, 128), lambda i: (i, 0))),
        out_shape=(jax.ShapeDtypeStruct((E // 8, 128), f32),
                   jax.ShapeDtypeStruct((E // 8, 128), f32)),
    )(nef8, w8a, w8b, b8)

    mesh = plsc.VectorSubcoreMesh(core_axis_name="c", subcore_axis_name="s")
    edge_fn = functools.partial(
        pl.kernel,
        out_type=(jax.ShapeDtypeStruct((NUM_CORES, NP, 32), f32),
                  jax.ShapeDtypeStruct((NUM_CORES, NP, 32), f32)),
        mesh=mesh,
        scratch_types=[
            pltpu.VMEM((BASE_ROWS + 1, CHUNK), i32),
            pltpu.VMEM((BASE_ROWS + 1, CHUNK), i32),
            pltpu.VMEM((2, CHUNK, 32), f32),
            pltpu.VMEM((2, CHUNK, 32), f32),
            pltpu.VMEM((16, 128), f32),
            pltpu.VMEM((16, 128), f32),
            pltpu.VMEM((16, 128), f32),
            pltpu.VMEM((16, 128), f32),
            pltpu.VMEM((CHUNK, 32), f32),
            pltpu.VMEM((CHUNK, 32), f32),
            pltpu.VMEM((16,), f32),
            pltpu.VMEM((16,), f32),
            pltpu.VMEM_SHARED((NP, 32), f32),
            pltpu.VMEM_SHARED((NP, 32), f32),
            pltpu.SemaphoreType.DMA,
            pltpu.SemaphoreType.DMA,
            pltpu.SemaphoreType.DMA,
            pltpu.SemaphoreType.DMA,
            pltpu.SemaphoreType.DMA,
            pltpu.SemaphoreType.DMA,
            pltpu.SemaphoreType.DMA,
        ],
        compiler_params=pltpu.CompilerParams(use_tc_tiling_on_sc=False),
    )(_edge_sc_kernel)
    sd_part, ss_part = edge_fn(ts, td, rea, reb, edge_index, zeros_acc,
                               w0v, b0v)

    b2o = b2_o2i.reshape(1, OUT_NF)
    w2f = W2_i2o[:, 1:17]
    b2f = b2_i2o[1:17].reshape(1, 16)
    b1r = b1_red.reshape(1, 16)
    b2r = b2_red.reshape(1, OUT_NF)
    RBLK = 2000
    out = pl.pallas_call(
        _fin_body,
        grid=(N // RBLK,),
        in_specs=[pl.BlockSpec((NUM_CORES, RBLK, 32), lambda i: (0, i, 0)),
                  pl.BlockSpec((NUM_CORES, RBLK, 32), lambda i: (0, i, 0)),
                  pl.BlockSpec((16, OUT_NF), lambda i: (0, 0)),
                  pl.BlockSpec((1, OUT_NF), lambda i: (0, 0)),
                  pl.BlockSpec((16, 16), lambda i: (0, 0)),
                  pl.BlockSpec((1, 16), lambda i: (0, 0)),
                  pl.BlockSpec((144, 16), lambda i: (0, 0)),
                  pl.BlockSpec((1, 16), lambda i: (0, 0)),
                  pl.BlockSpec((16, OUT_NF), lambda i: (0, 0)),
                  pl.BlockSpec((1, OUT_NF), lambda i: (0, 0))],
        out_specs=pl.BlockSpec((RBLK, OUT_NF), lambda i: (i, 0)),
        out_shape=jax.ShapeDtypeStruct((N, OUT_NF), f32),
    )(sd_part, ss_part, W2_o2i, b2o, w2f, b2f, W1_red, b1r, W2_red, b2r)
    return out
